# baseline (speedup 1.0000x reference)
import functools

import jax
import jax.numpy as jnp
from jax import lax
from jax.experimental import pallas as pl
from jax.experimental.pallas import tpu as pltpu
from jax.experimental.pallas import tpu_sc as plsc

HIDDEN = 128
CHUNK = 128
IB = 1024

_info = plsc.get_sparse_core_info()
_NC, _NS = _info.num_cores, _info.num_subcores
NW = _NC * _NS


def _make_gather(n_rows: int):
    n_per_w = n_rows // NW
    n_chunks = n_per_w // CHUNK
    n_batches = n_per_w // IB

    mesh = plsc.VectorSubcoreMesh(core_axis_name="c", subcore_axis_name="s")

    @functools.partial(
        pl.kernel,
        mesh=mesh,
        out_type=jax.ShapeDtypeStruct((n_rows * HIDDEN,), jnp.float32),
        scratch_types=[
            pltpu.VMEM_SHARED((_NS, IB), jnp.int32),
            pltpu.SMEM((IB,), jnp.int32),
            pltpu.VMEM_SHARED((_NS, 2, CHUNK * HIDDEN), jnp.float32),
            pltpu.SemaphoreType.DMA,
            pltpu.SemaphoreType.DMA,
            pltpu.SemaphoreType.DMA,
            pltpu.SemaphoreType.DMA,
        ],
    )
    def gather_kernel(idx_hbm, table_hbm, out_hbm, idx_sp, idx_s, sp_v, *sems):
        rs = sems[:2]
        ws = sems[2:]
        cid = lax.axis_index("c")
        sid = lax.axis_index("s")
        wid = sid * _NC + cid
        idx_base = wid * n_per_w
        row_base = wid * n_per_w
        sp = sp_v.at[sid]

        def issue_chunk(k, bb):
            c0 = (k % (IB // CHUNK)) * CHUNK

            def row_body(r, carry2):
                i = idx_s[c0 + r]
                off = pl.multiple_of(i * HIDDEN, 8)
                pltpu.async_copy(
                    table_hbm.at[pl.ds(off, HIDDEN)],
                    sp.at[bb, pl.ds(r * HIDDEN, HIDDEN)],
                    rs[bb],
                )
                return carry2

            lax.fori_loop(0, CHUNK, row_body, 0, unroll=8)

        def drain_rows(bb):
            pltpu.make_async_copy(
                table_hbm.at[pl.ds(0, CHUNK * HIDDEN)], sp.at[bb], rs[bb]
            ).wait()

        def out_start(k, bb):
            pltpu.async_copy(
                sp.at[bb],
                out_hbm.at[pl.ds((row_base + k * CHUNK) * HIDDEN, CHUNK * HIDDEN)],
                ws[bb],
            )

        def out_wait(k, bb):
            pltpu.make_async_copy(
                sp.at[bb],
                out_hbm.at[pl.ds((row_base + k * CHUNK) * HIDDEN, CHUNK * HIDDEN)],
                ws[bb],
            ).wait()

        def stage_idx(bt):
            pltpu.sync_copy(idx_hbm.at[pl.ds(idx_base + bt * IB, IB)], idx_sp.at[sid])
            pltpu.sync_copy(idx_sp.at[sid], idx_s)

        def visit(k, bb, do_wwait, do_prev):
            if do_wwait:
                out_wait(k - 2, bb)
            issue_chunk(k, bb)
            if do_prev:
                drain_rows(bb ^ 1)
                out_start(k - 1, bb ^ 1)

        stage_idx(0)
        visit(0, 0, False, False)
        visit(1, 1, False, True)

        def batch(bt, carry):
            stage_idx(bt)

            def chunk_body(j2, carry2):
                for bb in (0, 1):
                    k = bt * (IB // CHUNK) + 2 * j2 + bb
                    out_wait(k - 2, bb)
                    issue_chunk(k, bb)
                    drain_rows(bb ^ 1)
                    out_start(k - 1, bb ^ 1)
                return carry2

            lax.fori_loop(0, IB // CHUNK // 2, chunk_body, 0, unroll=False)
            return carry

        def chunk_mid(j2, carry2):
            for bb in (0, 1):
                j = 2 * j2 + bb
                out_wait(j - 2, bb)
                issue_chunk(j, bb)
                drain_rows(bb ^ 1)
                out_start(j - 1, bb ^ 1)
            return carry2

        lax.fori_loop(1, IB // CHUNK // 2, chunk_mid, 0, unroll=False)
        lax.fori_loop(1, n_batches, batch, 0, unroll=False)

        kl = n_chunks - 1
        drain_rows(kl % 2)
        out_start(kl, kl % 2)
        out_wait(kl - 1, (kl - 1) % 2)
        out_wait(kl, kl % 2)

    return gather_kernel


def kernel(input_ids, weight):
    b, s = input_ids.shape
    n_rows = b * s
    idx = input_ids.reshape(-1).astype(jnp.int32)
    table_flat = weight.reshape(-1)
    out = _make_gather(n_rows)(idx, table_flat)
    return out.reshape(b, s, HIDDEN)


# speedup vs baseline: 5.3026x; 5.3026x over previous
import functools

import jax
import jax.numpy as jnp
from jax import lax
from jax.experimental import pallas as pl
from jax.experimental.pallas import tpu as pltpu
from jax.experimental.pallas import tpu_sc as plsc

HIDDEN = 128
CHUNK = 128
NBUF = 4
LOOK = 2
SB = 2

_info = plsc.get_sparse_core_info()
_NC, _NS = _info.num_cores, _info.num_subcores
NW = _NC * _NS


def _make_gather(n_rows: int):
    n_per_w = n_rows // NW
    n_chunks = n_per_w // CHUNK
    n_blocks = n_chunks // NBUF
    assert n_chunks % NBUF == 0 and n_blocks >= 3

    mesh = plsc.VectorSubcoreMesh(core_axis_name="c", subcore_axis_name="s")

    @functools.partial(
        pl.kernel,
        mesh=mesh,
        out_type=jax.ShapeDtypeStruct((n_rows, HIDDEN), jnp.float32),
        scratch_types=[
            pltpu.VMEM((n_chunks, CHUNK), jnp.int32),
            pltpu.VMEM((NBUF, CHUNK, HIDDEN), jnp.float32),
            pltpu.VMEM_SHARED((_NS, SB, CHUNK, HIDDEN), jnp.float32),
        ]
        + [pltpu.SemaphoreType.DMA] * (2 * NBUF + SB),
    )
    def gather_kernel(idx_hbm, table_hbm, out_hbm, idx_v, rows_v, sp_v, *sems):
        gs, cs, ds = sems[:NBUF], sems[NBUF : 2 * NBUF], sems[2 * NBUF :]
        cid = lax.axis_index("c")
        sid = lax.axis_index("s")
        wid = sid * _NC + cid
        pltpu.sync_copy(idx_hbm.at[pl.ds(wid * n_chunks, n_chunks)], idx_v)
        row_base = wid * n_per_w

        def g_start(k, b):
            pltpu.async_copy(table_hbm.at[idx_v.at[k]], rows_v.at[b], gs[b])

        def g_wait(k, b):
            pltpu.make_async_copy(
                table_hbm.at[idx_v.at[k]], rows_v.at[b], gs[b]
            ).wait()

        def c_start(b):
            pltpu.async_copy(rows_v.at[b], sp_v.at[sid, b % SB], cs[b])

        def c_wait(b):
            pltpu.make_async_copy(rows_v.at[b], sp_v.at[sid, b % SB], cs[b]).wait()

        def d_start(k, b):
            pltpu.async_copy(
                sp_v.at[sid, b % SB],
                out_hbm.at[pl.ds(row_base + k * CHUNK, CHUNK)],
                ds[b % SB],
            )

        def d_wait(k, b):
            pltpu.make_async_copy(
                sp_v.at[sid, b % SB],
                out_hbm.at[pl.ds(row_base + k * CHUNK, CHUNK)],
                ds[b % SB],
            ).wait()

        def visit(k, b, do_dwait, do_gstart):
            g_wait(k, b)
            if do_dwait:
                d_wait(k - SB, b)
            c_start(b)
            c_wait(b)
            if do_gstart:
                g_start(k + LOOK, (b + LOOK) % NBUF)
            d_start(k, b)

        for k in range(LOOK):
            g_start(k, k)
        for b in range(NBUF):
            visit(b, b, do_dwait=(b >= SB), do_gstart=True)

        def block(jj, carry):
            for b in range(NBUF):
                visit(NBUF * jj + b, b, do_dwait=True, do_gstart=True)
            return carry

        lax.fori_loop(1, n_blocks - 1, block, 0, unroll=False)

        kl = n_chunks - NBUF
        for b in range(NBUF):
            visit(kl + b, b, do_dwait=True, do_gstart=(kl + b + LOOK < n_chunks))
        for b in range(NBUF - SB, NBUF):
            d_wait(kl + b, b)

    return gather_kernel


def kernel(input_ids, weight):
    b, s = input_ids.shape
    n_rows = b * s
    idx = input_ids.reshape(n_rows // CHUNK, CHUNK).astype(jnp.int32)
    out = _make_gather(n_rows)(idx, weight)
    return out.reshape(b, s, HIDDEN)
